# Initial kernel scaffold; baseline (speedup 1.0000x reference)
import jax
import jax.numpy as jnp
from jax.experimental import pallas as pl


def kernel(child_feats, child_exists, edge_indices, W_m1a, b_m1a, W_m1b, b_m1b, W_skip10, b_skip10, W_m2, b_m2, W_child, b_child, W_ne0, b_ne0, W_ne1, b_ne1, W_skipobj, b_skipobj, W_second, b_second):
    raise NotImplementedError("write your pallas kernel here")


# speedup vs baseline: 1.5648x; 1.5648x over previous
import functools

import jax
import jax.numpy as jnp
from jax import lax
from jax.experimental import pallas as pl
from jax.experimental.pallas import tpu as pltpu
from jax.experimental.pallas import tpu_sc as plsc

_NEG = -3.0e38
_BN = 2048


def _lr(x):
    return jnp.where(x > 0, x, 0.1 * x)


def _k1_body(boxT, semT, ceT, Wm1aT, bm1a, Wm1bT, bm1b, Ws10T, bs10,
             Wm2T, bm2, WchT, bch, WsoT, bso, WsrcT, WdstT,
             A1T_o, B1T_o, p1_o, sk_o):
    j = pl.program_id(0)
    box = boxT[...]
    h = _lr(Wm1aT[...] @ box + bm1a[...])
    h = _lr(Wm1bT[...] @ h + bm1b[...])
    enc = _lr(Ws10T[...] @ box + bs10[...] + Wm2T[...] @ h + bm2[...])
    cfeat = jnp.concatenate([enc, semT[...]], axis=0)
    ce = ceT[...]
    skip = jnp.where(ce > 0, (WsoT[...] @ cfeat + bso[...]) * ce, _NEG)
    cf1 = jnp.maximum(WchT[...] @ cfeat + bch[...], 0.0) * ce
    A1T_o[...] = WsrcT[...] @ cf1
    B1T_o[...] = WdstT[...] @ cf1
    pb = jnp.max(cf1, axis=1, keepdims=True)
    sb = jnp.max(skip, axis=1, keepdims=True)

    @pl.when(j == 0)
    def _():
        p1_o[...] = pb
        sk_o[...] = sb

    @pl.when(j > 0)
    def _():
        p1_o[...] = jnp.maximum(p1_o[...], pb)
        sk_o[...] = jnp.maximum(sk_o[...], sb)


def _run_k1(boxT, semT, ceT, ws, N):
    ng = N // _BN
    col = lambda r: pl.BlockSpec((r, _BN), lambda j: (0, j))
    full = lambda a: pl.BlockSpec(a.shape, lambda j: (0, 0))
    out_col = pl.BlockSpec((128, _BN), lambda j: (0, j))
    acc = pl.BlockSpec((128, 1), lambda j: (0, 0))
    return pl.pallas_call(
        _k1_body,
        grid=(ng,),
        in_specs=[col(10), col(24), col(1)] + [full(w) for w in ws],
        out_specs=[out_col, out_col, acc, acc],
        out_shape=[
            jax.ShapeDtypeStruct((128, N), jnp.float32),
            jax.ShapeDtypeStruct((128, N), jnp.float32),
            jax.ShapeDtypeStruct((128, 1), jnp.float32),
            jax.ShapeDtypeStruct((128, 1), jnp.float32),
        ],
    )(boxT, semT, ceT, *ws)


def _k2_body(AT, MT, bne, WsrcT, WdstT, A2T_o, B2T_o, p_o):
    j = pl.program_id(0)
    cf = jnp.maximum(AT[...] + bne[...] + MT[...], 0.0)
    A2T_o[...] = WsrcT[...] @ cf
    B2T_o[...] = WdstT[...] @ cf
    pb = jnp.max(cf, axis=1, keepdims=True)

    @pl.when(j == 0)
    def _():
        p_o[...] = pb

    @pl.when(j > 0)
    def _():
        p_o[...] = jnp.maximum(p_o[...], pb)


def _run_k2(AT, MT, bne, WsrcT, WdstT, N):
    ng = N // _BN
    col = pl.BlockSpec((128, _BN), lambda j: (0, j))
    full = lambda a: pl.BlockSpec(a.shape, lambda j: (0, 0))
    acc = pl.BlockSpec((128, 1), lambda j: (0, 0))
    return pl.pallas_call(
        _k2_body,
        grid=(ng,),
        in_specs=[col, col, full(bne), full(WsrcT), full(WdstT)],
        out_specs=[col, col, acc],
        out_shape=[
            jax.ShapeDtypeStruct((128, N), jnp.float32),
            jax.ShapeDtypeStruct((128, N), jnp.float32),
            jax.ShapeDtypeStruct((128, 1), jnp.float32),
        ],
    )(AT, MT, bne, WsrcT, WdstT)


def _k3_body(AT, MT, bne, p1, p2, skmax, WsecT, bsec, out_o, p3s):
    j = pl.program_id(0)
    ng = pl.num_programs(0)
    cf = jnp.maximum(AT[...] + bne[...] + MT[...], 0.0)
    pb = jnp.max(cf, axis=1, keepdims=True)

    @pl.when(j == 0)
    def _():
        p3s[...] = pb

    @pl.when(j > 0)
    def _():
        p3s[...] = jnp.maximum(p3s[...], pb)

    @pl.when(j == ng - 1)
    def _():
        parent = jnp.concatenate([p1[...], p2[...], p3s[...]], axis=0)
        out_o[...] = _lr(_lr(skmax[...]) + WsecT[...] @ parent + bsec[...])


def _run_k3(AT, MT, bne, p1, p2, skmax, WsecT, bsec, N):
    ng = N // _BN
    col = pl.BlockSpec((128, _BN), lambda j: (0, j))
    full = lambda a: pl.BlockSpec(a.shape, lambda j: (0, 0))
    acc = pl.BlockSpec((128, 1), lambda j: (0, 0))
    return pl.pallas_call(
        _k3_body,
        grid=(ng,),
        in_specs=[col, col, full(bne), full(p1), full(p2), full(skmax),
                  full(WsecT), full(bsec)],
        out_specs=acc,
        out_shape=jax.ShapeDtypeStruct((128, 1), jnp.float32),
        scratch_shapes=[pltpu.VMEM((128, 1), jnp.float32)],
    )(AT, MT, bne, p1, p2, skmax, WsecT, bsec)


def _segmax_sc(bT_flat, src, dst, N):
    E = src.shape[0]
    NC, NS = 2, 16
    NW = NC * NS
    FPW = 128 // NW
    CH = 2000
    G = CH // 16
    NCH = E // CH
    assert CH * NCH == E and FPW * NW == 128

    mesh = plsc.VectorSubcoreMesh(core_axis_name="c", subcore_axis_name="s",
                                  num_cores=NC, num_subcores=NS)

    @functools.partial(
        pl.kernel,
        out_type=jax.ShapeDtypeStruct((128 * N,), jnp.float32),
        mesh=mesh,
        scratch_types=[
            pltpu.VMEM((FPW * N,), jnp.float32),
            pltpu.VMEM((FPW * N,), jnp.float32),
            pltpu.VMEM((CH,), jnp.int32),
            pltpu.VMEM((CH,), jnp.int32),
            pltpu.VMEM((N,), jnp.int32),
        ],
        compiler_params=pltpu.CompilerParams(needs_layout_passes=False),
    )
    def k(b_hbm, s_hbm, d_hbm, out_hbm, b_v, m_v, s_v, d_v, dup_v):
        wid = lax.axis_index("s") * NC + lax.axis_index("c")
        base = wid * (FPW * N)
        pltpu.sync_copy(b_hbm.at[pl.ds(base, FPW * N)], b_v)

        neg = jnp.full((16,), _NEG, jnp.float32)

        def init_body(i, c):
            m_v[pl.ds(i * 16, 16)] = neg
            return c

        lax.fori_loop(0, (FPW * N) // 16, init_body, 0)

        lanes = lax.broadcasted_iota(jnp.int32, (16,), 0)

        def chunk_body(c, carry):
            pltpu.sync_copy(s_hbm.at[pl.ds(c * CH, CH)], s_v)
            pltpu.sync_copy(d_hbm.at[pl.ds(c * CH, CH)], d_v)

            def group_body(g, carry2):
                s16 = s_v[pl.ds(g * 16, 16)]
                d16 = d_v[pl.ds(g * 16, 16)]
                plsc.store_scatter(dup_v, [s16], lanes)
                rb = plsc.load_gather(dup_v, [s16])
                hasdup = jnp.any(rb != lanes)

                def fast(_):
                    for f in range(FPW):
                        v = plsc.load_gather(b_v, [d16 + f * N])
                        old = plsc.load_gather(m_v, [s16 + f * N])
                        plsc.store_scatter(m_v, [s16 + f * N],
                                           jnp.maximum(old, v))
                    return 0

                def slow(_):
                    vs = [plsc.load_gather(b_v, [d16 + f * N])
                          for f in range(FPW)]

                    def lane_body(l, c3):
                        m = lanes == l
                        for f in range(FPW):
                            old = plsc.load_gather(m_v, [s16 + f * N], mask=m)
                            plsc.store_scatter(m_v, [s16 + f * N],
                                               jnp.maximum(old, vs[f]),
                                               mask=m)
                        return c3

                    return lax.fori_loop(0, 16, lane_body, 0)

                lax.cond(hasdup, slow, fast, 0)
                return carry2

            lax.fori_loop(0, G, group_body, 0)
            return carry

        lax.fori_loop(0, NCH, chunk_body, 0)
        pltpu.sync_copy(m_v, out_hbm.at[pl.ds(base, FPW * N)])

    return k(bT_flat, src, dst)


def kernel(child_feats, child_exists, edge_indices,
           W_m1a, b_m1a, W_m1b, b_m1b, W_skip10, b_skip10, W_m2, b_m2,
           W_child, b_child, W_ne0, b_ne0, W_ne1, b_ne1,
           W_skipobj, b_skipobj, W_second, b_second):
    N = child_feats.shape[1]
    H = 128
    Np = ((N + _BN - 1) // _BN) * _BN

    xT = child_feats[0].T
    pad = ((0, 0), (0, Np - N))
    boxT = jnp.pad(xT[:10], pad)
    semT = jnp.pad(xT[10:], pad)
    ceT = jnp.pad(child_exists[0].T, pad)
    src = edge_indices[0, :, 0]
    dst = edge_indices[0, :, 1]

    colv = lambda b: b.reshape(-1, 1)
    ws = [W_m1a.T, colv(b_m1a), W_m1b.T, colv(b_m1b),
          W_skip10.T, colv(b_skip10), W_m2.T, colv(b_m2),
          W_child.T, colv(b_child), W_skipobj.T, colv(b_skipobj),
          W_ne0[:H].T, W_ne0[H:].T]

    A1T, B1T, p1, skmax = _run_k1(boxT, semT, ceT, ws, Np)

    M1T = _segmax_sc(B1T.reshape(-1), src, dst, Np).reshape(H, Np)
    A2T, B2T, p2 = _run_k2(A1T, M1T, colv(b_ne0),
                           W_ne1[:H].T, W_ne1[H:].T, Np)
    M2T = _segmax_sc(B2T.reshape(-1), src, dst, Np).reshape(H, Np)
    outT = _run_k3(A2T, M2T, colv(b_ne1), p1, p2, skmax,
                   W_second.T, colv(b_second), Np)
    return outT.reshape(1, H)


# speedup vs baseline: 2.9168x; 1.8640x over previous
import functools

import jax
import jax.numpy as jnp
from jax import lax
from jax.experimental import pallas as pl
from jax.experimental.pallas import tpu as pltpu
from jax.experimental.pallas import tpu_sc as plsc

_NEG = -3.0e38
_BN = 2048


def _lr(x):
    return jnp.where(x > 0, x, 0.1 * x)


def _k1_body(boxT, semT, ceT, Wm1aT, bm1a, Wm1bT, bm1b, Ws10T, bs10,
             Wm2T, bm2, WchT, bch, WsoT, bso, WsrcT, WdstT,
             A1T_o, B1T_o, p1_o, sk_o):
    j = pl.program_id(0)
    box = boxT[...]
    h = _lr(Wm1aT[...] @ box + bm1a[...])
    h = _lr(Wm1bT[...] @ h + bm1b[...])
    enc = _lr(Ws10T[...] @ box + bs10[...] + Wm2T[...] @ h + bm2[...])
    cfeat = jnp.concatenate([enc, semT[...]], axis=0)
    ce = ceT[...]
    skip = jnp.where(ce > 0, (WsoT[...] @ cfeat + bso[...]) * ce, _NEG)
    cf1 = jnp.maximum(WchT[...] @ cfeat + bch[...], 0.0) * ce
    A1T_o[...] = WsrcT[...] @ cf1
    B1T_o[...] = WdstT[...] @ cf1
    pb = jnp.max(cf1, axis=1, keepdims=True)
    sb = jnp.max(skip, axis=1, keepdims=True)

    @pl.when(j == 0)
    def _():
        p1_o[...] = pb
        sk_o[...] = sb

    @pl.when(j > 0)
    def _():
        p1_o[...] = jnp.maximum(p1_o[...], pb)
        sk_o[...] = jnp.maximum(sk_o[...], sb)


def _run_k1(boxT, semT, ceT, ws, N):
    ng = N // _BN
    col = lambda r: pl.BlockSpec((r, _BN), lambda j: (0, j))
    full = lambda a: pl.BlockSpec(a.shape, lambda j: (0, 0))
    out_col = pl.BlockSpec((128, _BN), lambda j: (0, j))
    acc = pl.BlockSpec((128, 1), lambda j: (0, 0))
    return pl.pallas_call(
        _k1_body,
        grid=(ng,),
        in_specs=[col(10), col(24), col(1)] + [full(w) for w in ws],
        out_specs=[out_col, out_col, acc, acc],
        out_shape=[
            jax.ShapeDtypeStruct((128, N), jnp.float32),
            jax.ShapeDtypeStruct((128, N), jnp.float32),
            jax.ShapeDtypeStruct((128, 1), jnp.float32),
            jax.ShapeDtypeStruct((128, 1), jnp.float32),
        ],
    )(boxT, semT, ceT, *ws)


def _k2_body(AT, MT, bne, WsrcT, WdstT, A2T_o, B2T_o, p_o):
    j = pl.program_id(0)
    cf = jnp.maximum(AT[...] + bne[...] + MT[...], 0.0)
    A2T_o[...] = WsrcT[...] @ cf
    B2T_o[...] = WdstT[...] @ cf
    pb = jnp.max(cf, axis=1, keepdims=True)

    @pl.when(j == 0)
    def _():
        p_o[...] = pb

    @pl.when(j > 0)
    def _():
        p_o[...] = jnp.maximum(p_o[...], pb)


def _run_k2(AT, MT, bne, WsrcT, WdstT, N):
    ng = N // _BN
    col = pl.BlockSpec((128, _BN), lambda j: (0, j))
    full = lambda a: pl.BlockSpec(a.shape, lambda j: (0, 0))
    acc = pl.BlockSpec((128, 1), lambda j: (0, 0))
    return pl.pallas_call(
        _k2_body,
        grid=(ng,),
        in_specs=[col, col, full(bne), full(WsrcT), full(WdstT)],
        out_specs=[col, col, acc],
        out_shape=[
            jax.ShapeDtypeStruct((128, N), jnp.float32),
            jax.ShapeDtypeStruct((128, N), jnp.float32),
            jax.ShapeDtypeStruct((128, 1), jnp.float32),
        ],
    )(AT, MT, bne, WsrcT, WdstT)


def _k3_body(AT, MT, bne, p1, p2, skmax, WsecT, bsec, out_o, p3s):
    j = pl.program_id(0)
    ng = pl.num_programs(0)
    cf = jnp.maximum(AT[...] + bne[...] + MT[...], 0.0)
    pb = jnp.max(cf, axis=1, keepdims=True)

    @pl.when(j == 0)
    def _():
        p3s[...] = pb

    @pl.when(j > 0)
    def _():
        p3s[...] = jnp.maximum(p3s[...], pb)

    @pl.when(j == ng - 1)
    def _():
        parent = jnp.concatenate([p1[...], p2[...], p3s[...]], axis=0)
        out_o[...] = _lr(_lr(skmax[...]) + WsecT[...] @ parent + bsec[...])


def _run_k3(AT, MT, bne, p1, p2, skmax, WsecT, bsec, N):
    ng = N // _BN
    col = pl.BlockSpec((128, _BN), lambda j: (0, j))
    full = lambda a: pl.BlockSpec(a.shape, lambda j: (0, 0))
    acc = pl.BlockSpec((128, 1), lambda j: (0, 0))
    return pl.pallas_call(
        _k3_body,
        grid=(ng,),
        in_specs=[col, col, full(bne), full(p1), full(p2), full(skmax),
                  full(WsecT), full(bsec)],
        out_specs=acc,
        out_shape=jax.ShapeDtypeStruct((128, 1), jnp.float32),
        scratch_shapes=[pltpu.VMEM((128, 1), jnp.float32)],
    )(AT, MT, bne, p1, p2, skmax, WsecT, bsec)


def _segmax_sc(bT_flat, src, dst, N):
    E = src.shape[0]
    NC, NS = 2, 16
    NW = NC * NS
    FPW = 128 // NW
    CH = 1600
    G = CH // 16
    NCH = E // CH
    assert CH * NCH == E and FPW * NW == 128
    assert G % 2 == 0 and NCH % 2 == 0

    mesh = plsc.VectorSubcoreMesh(core_axis_name="c", subcore_axis_name="s",
                                  num_cores=NC, num_subcores=NS)

    @functools.partial(
        pl.kernel,
        out_type=jax.ShapeDtypeStruct((128 * N,), jnp.float32),
        mesh=mesh,
        scratch_types=[
            pltpu.VMEM((FPW * N,), jnp.float32),
            pltpu.VMEM((FPW * N,), jnp.float32),
            pltpu.VMEM((FPW * N,), jnp.float32),
            pltpu.VMEM((CH,), jnp.int32),
            pltpu.VMEM((CH,), jnp.int32),
            pltpu.VMEM((CH,), jnp.int32),
            pltpu.VMEM((CH,), jnp.int32),
            pltpu.SemaphoreType.DMA,
            pltpu.SemaphoreType.DMA,
        ],
        compiler_params=pltpu.CompilerParams(needs_layout_passes=False),
    )
    def k(b_hbm, s_hbm, d_hbm, out_hbm, b_v, m_a, m_b,
          s0, d0, s1, d1, sem0, sem1):
        wid = lax.axis_index("s") * NC + lax.axis_index("c")
        base = wid * (FPW * N)
        pltpu.sync_copy(b_hbm.at[pl.ds(base, FPW * N)], b_v)

        neg = jnp.full((16,), _NEG, jnp.float32)

        def init_body(i, c):
            m_a[pl.ds(i * 16, 16)] = neg
            m_b[pl.ds(i * 16, 16)] = neg
            return c

        lax.fori_loop(0, (FPW * N) // 16, init_body, 0)

        lanes = lax.broadcasted_iota(jnp.int32, (16,), 0)

        def rmw(m_v, s16, d16):
            for f in range(FPW):
                v = plsc.load_gather(b_v, [d16 + f * N])
                old = plsc.load_gather(m_v, [s16 + f * N])
                plsc.store_scatter(m_v, [s16 + f * N], jnp.maximum(old, v))

        def fixup(m_v, s16, d16):
            vs = [plsc.load_gather(b_v, [d16 + f * N]) for f in range(FPW)]

            def lane_body(l, c3):
                m = lanes == l
                for f in range(FPW):
                    old = plsc.load_gather(m_v, [s16 + f * N], mask=m)
                    plsc.store_scatter(m_v, [s16 + f * N],
                                       jnp.maximum(old, vs[f]), mask=m)
                return c3

            lax.fori_loop(0, 16, lane_body, 0)

        def process(s_ref, d_ref):
            def pair_body(p, carry):
                sa = s_ref[pl.ds(p * 32, 16)]
                da = d_ref[pl.ds(p * 32, 16)]
                sb = s_ref[pl.ds(p * 32 + 16, 16)]
                db = d_ref[pl.ds(p * 32 + 16, 16)]
                _, la = plsc.scan_count(sa)
                _, lb = plsc.scan_count(sb)
                rmw(m_a, sa, da)
                rmw(m_b, sb, db)
                anydup = jnp.any(jnp.logical_not(la & lb))

                @pl.when(anydup)
                def _():
                    fixup(m_a, sa, da)
                    fixup(m_b, sb, db)

                return carry

            lax.fori_loop(0, G // 2, pair_body, 0)

        def start_chunk(c, s_buf, d_buf, sem):
            pltpu.make_async_copy(s_hbm.at[pl.ds(c * CH, CH)], s_buf,
                                  sem).start()
            pltpu.make_async_copy(d_hbm.at[pl.ds(c * CH, CH)], d_buf,
                                  sem).start()

        def wait_chunk(c, s_buf, d_buf, sem):
            pltpu.make_async_copy(s_hbm.at[pl.ds(c * CH, CH)], s_buf,
                                  sem).wait()
            pltpu.make_async_copy(d_hbm.at[pl.ds(c * CH, CH)], d_buf,
                                  sem).wait()

        start_chunk(0, s0, d0, sem0)

        def outer(i, carry):
            c0 = 2 * i
            start_chunk(c0 + 1, s1, d1, sem1)
            wait_chunk(c0, s0, d0, sem0)
            process(s0, d0)

            @pl.when(c0 + 2 < NCH)
            def _():
                start_chunk(c0 + 2, s0, d0, sem0)

            wait_chunk(c0 + 1, s1, d1, sem1)
            process(s1, d1)
            return carry

        lax.fori_loop(0, NCH // 2, outer, 0)

        def merge_body(i, c):
            m_a[pl.ds(i * 16, 16)] = jnp.maximum(m_a[pl.ds(i * 16, 16)],
                                                 m_b[pl.ds(i * 16, 16)])
            return c

        lax.fori_loop(0, (FPW * N) // 16, merge_body, 0)
        pltpu.sync_copy(m_a, out_hbm.at[pl.ds(base, FPW * N)])

    return k(bT_flat, src, dst)


def kernel(child_feats, child_exists, edge_indices,
           W_m1a, b_m1a, W_m1b, b_m1b, W_skip10, b_skip10, W_m2, b_m2,
           W_child, b_child, W_ne0, b_ne0, W_ne1, b_ne1,
           W_skipobj, b_skipobj, W_second, b_second):
    N = child_feats.shape[1]
    H = 128
    Np = ((N + _BN - 1) // _BN) * _BN

    xT = child_feats[0].T
    pad = ((0, 0), (0, Np - N))
    boxT = jnp.pad(xT[:10], pad)
    semT = jnp.pad(xT[10:], pad)
    ceT = jnp.pad(child_exists[0].T, pad)
    src = edge_indices[0, :, 0]
    dst = edge_indices[0, :, 1]

    colv = lambda b: b.reshape(-1, 1)
    ws = [W_m1a.T, colv(b_m1a), W_m1b.T, colv(b_m1b),
          W_skip10.T, colv(b_skip10), W_m2.T, colv(b_m2),
          W_child.T, colv(b_child), W_skipobj.T, colv(b_skipobj),
          W_ne0[:H].T, W_ne0[H:].T]

    A1T, B1T, p1, skmax = _run_k1(boxT, semT, ceT, ws, Np)

    M1T = _segmax_sc(B1T.reshape(-1), src, dst, Np).reshape(H, Np)
    A2T, B2T, p2 = _run_k2(A1T, M1T, colv(b_ne0),
                           W_ne1[:H].T, W_ne1[H:].T, Np)
    M2T = _segmax_sc(B2T.reshape(-1), src, dst, Np).reshape(H, Np)
    outT = _run_k3(A2T, M2T, colv(b_ne1), p1, p2, skmax,
                   W_second.T, colv(b_second), Np)
    return outT.reshape(1, H)


# speedup vs baseline: 3.0411x; 1.0426x over previous
import functools

import jax
import jax.numpy as jnp
from jax import lax
from jax.experimental import pallas as pl
from jax.experimental.pallas import tpu as pltpu
from jax.experimental.pallas import tpu_sc as plsc

_NEG = -3.0e38
_BN = 2048


def _lr(x):
    return jnp.where(x > 0, x, 0.1 * x)


def _k1_body(boxT, semT, ceT, Wm1aT, bm1a, Wm1bT, bm1b, Ws10T, bs10,
             Wm2T, bm2, WchT, bch, WsoT, bso, WsrcT, WdstT,
             A1T_o, B1T_o, p1_o, sk_o):
    j = pl.program_id(0)
    box = boxT[...]
    h = _lr(Wm1aT[...] @ box + bm1a[...])
    h = _lr(Wm1bT[...] @ h + bm1b[...])
    enc = _lr(Ws10T[...] @ box + bs10[...] + Wm2T[...] @ h + bm2[...])
    cfeat = jnp.concatenate([enc, semT[...]], axis=0)
    ce = ceT[...]
    skip = jnp.where(ce > 0, (WsoT[...] @ cfeat + bso[...]) * ce, _NEG)
    cf1 = jnp.maximum(WchT[...] @ cfeat + bch[...], 0.0) * ce
    A1T_o[...] = WsrcT[...] @ cf1
    B1T_o[...] = WdstT[...] @ cf1
    pb = jnp.max(cf1, axis=1, keepdims=True)
    sb = jnp.max(skip, axis=1, keepdims=True)

    @pl.when(j == 0)
    def _():
        p1_o[...] = pb
        sk_o[...] = sb

    @pl.when(j > 0)
    def _():
        p1_o[...] = jnp.maximum(p1_o[...], pb)
        sk_o[...] = jnp.maximum(sk_o[...], sb)


def _run_k1(boxT, semT, ceT, ws, N):
    ng = N // _BN
    col = lambda r: pl.BlockSpec((r, _BN), lambda j: (0, j))
    full = lambda a: pl.BlockSpec(a.shape, lambda j: (0, 0))
    out_col = pl.BlockSpec((128, _BN), lambda j: (0, j))
    acc = pl.BlockSpec((128, 1), lambda j: (0, 0))
    return pl.pallas_call(
        _k1_body,
        grid=(ng,),
        in_specs=[col(10), col(24), col(1)] + [full(w) for w in ws],
        out_specs=[out_col, out_col, acc, acc],
        out_shape=[
            jax.ShapeDtypeStruct((128, N), jnp.float32),
            jax.ShapeDtypeStruct((128, N), jnp.float32),
            jax.ShapeDtypeStruct((128, 1), jnp.float32),
            jax.ShapeDtypeStruct((128, 1), jnp.float32),
        ],
    )(boxT, semT, ceT, *ws)


def _k2_body(AT, MT, bne, WsrcT, WdstT, A2T_o, B2T_o, p_o):
    j = pl.program_id(0)
    cf = jnp.maximum(AT[...] + bne[...] + MT[...], 0.0)
    A2T_o[...] = WsrcT[...] @ cf
    B2T_o[...] = WdstT[...] @ cf
    pb = jnp.max(cf, axis=1, keepdims=True)

    @pl.when(j == 0)
    def _():
        p_o[...] = pb

    @pl.when(j > 0)
    def _():
        p_o[...] = jnp.maximum(p_o[...], pb)


def _run_k2(AT, MT, bne, WsrcT, WdstT, N):
    ng = N // _BN
    col = pl.BlockSpec((128, _BN), lambda j: (0, j))
    full = lambda a: pl.BlockSpec(a.shape, lambda j: (0, 0))
    acc = pl.BlockSpec((128, 1), lambda j: (0, 0))
    return pl.pallas_call(
        _k2_body,
        grid=(ng,),
        in_specs=[col, col, full(bne), full(WsrcT), full(WdstT)],
        out_specs=[col, col, acc],
        out_shape=[
            jax.ShapeDtypeStruct((128, N), jnp.float32),
            jax.ShapeDtypeStruct((128, N), jnp.float32),
            jax.ShapeDtypeStruct((128, 1), jnp.float32),
        ],
    )(AT, MT, bne, WsrcT, WdstT)


def _k3_body(AT, MT, bne, p1, p2, skmax, WsecT, bsec, out_o, p3s):
    j = pl.program_id(0)
    ng = pl.num_programs(0)
    cf = jnp.maximum(AT[...] + bne[...] + MT[...], 0.0)
    pb = jnp.max(cf, axis=1, keepdims=True)

    @pl.when(j == 0)
    def _():
        p3s[...] = pb

    @pl.when(j > 0)
    def _():
        p3s[...] = jnp.maximum(p3s[...], pb)

    @pl.when(j == ng - 1)
    def _():
        parent = jnp.concatenate([p1[...], p2[...], p3s[...]], axis=0)
        out_o[...] = _lr(_lr(skmax[...]) + WsecT[...] @ parent + bsec[...])


def _run_k3(AT, MT, bne, p1, p2, skmax, WsecT, bsec, N):
    ng = N // _BN
    col = pl.BlockSpec((128, _BN), lambda j: (0, j))
    full = lambda a: pl.BlockSpec(a.shape, lambda j: (0, 0))
    acc = pl.BlockSpec((128, 1), lambda j: (0, 0))
    return pl.pallas_call(
        _k3_body,
        grid=(ng,),
        in_specs=[col, col, full(bne), full(p1), full(p2), full(skmax),
                  full(WsecT), full(bsec)],
        out_specs=acc,
        out_shape=jax.ShapeDtypeStruct((128, 1), jnp.float32),
        scratch_shapes=[pltpu.VMEM((128, 1), jnp.float32)],
    )(AT, MT, bne, p1, p2, skmax, WsecT, bsec)


def _segmax_sc(bT_flat, src, dst, N):
    E = src.shape[0]
    NC, NS = 2, 16
    NW = NC * NS
    FPW = 128 // NW
    CH = 1600
    G = CH // 16
    NCH = E // CH
    assert CH * NCH == E and FPW * NW == 128
    assert G % 2 == 0 and NCH % 2 == 0

    mesh = plsc.VectorSubcoreMesh(core_axis_name="c", subcore_axis_name="s",
                                  num_cores=NC, num_subcores=NS)

    @functools.partial(
        pl.kernel,
        out_type=jax.ShapeDtypeStruct((128 * N,), jnp.float32),
        mesh=mesh,
        scratch_types=(
            [pltpu.VMEM((N,), jnp.float32)] * FPW +
            [pltpu.VMEM((N,), jnp.float32)] * FPW +
            [pltpu.VMEM((N,), jnp.float32)] * FPW +
            [
                pltpu.VMEM((CH,), jnp.int32),
                pltpu.VMEM((CH,), jnp.int32),
                pltpu.VMEM((CH,), jnp.int32),
                pltpu.VMEM((CH,), jnp.int32),
                pltpu.SemaphoreType.DMA,
                pltpu.SemaphoreType.DMA,
            ]
        ),
        compiler_params=pltpu.CompilerParams(needs_layout_passes=False),
    )
    def k(b_hbm, s_hbm, d_hbm, out_hbm,
          b0, b1, b2, b3, ma0, ma1, ma2, ma3, mb0, mb1, mb2, mb3,
          s0, d0, s1, d1, sem0, sem1):
        bs = [b0, b1, b2, b3]
        mas = [ma0, ma1, ma2, ma3]
        mbs = [mb0, mb1, mb2, mb3]
        wid = lax.axis_index("s") * NC + lax.axis_index("c")
        base = wid * (FPW * N)
        for f in range(FPW):
            pltpu.sync_copy(b_hbm.at[pl.ds(base + f * N, N)], bs[f])

        neg = jnp.full((16,), _NEG, jnp.float32)

        def init_body(i, c):
            for f in range(FPW):
                mas[f][pl.ds(i * 16, 16)] = neg
                mbs[f][pl.ds(i * 16, 16)] = neg
            return c

        lax.fori_loop(0, N // 16, init_body, 0)

        lanes = lax.broadcasted_iota(jnp.int32, (16,), 0)

        def rmw(ms, s16, d16):
            for f in range(FPW):
                v = plsc.load_gather(bs[f], [d16])
                old = plsc.load_gather(ms[f], [s16])
                plsc.store_scatter(ms[f], [s16], jnp.maximum(old, v))

        def fixup(ms, s16, d16):
            vs = [plsc.load_gather(bs[f], [d16]) for f in range(FPW)]

            def lane_body(l, c3):
                m = lanes == l
                for f in range(FPW):
                    old = plsc.load_gather(ms[f], [s16], mask=m)
                    plsc.store_scatter(ms[f], [s16],
                                       jnp.maximum(old, vs[f]), mask=m)
                return c3

            lax.fori_loop(0, 16, lane_body, 0)

        def process(s_ref, d_ref):
            def pair_body(p, carry):
                sa = s_ref[pl.ds(p * 32, 16)]
                da = d_ref[pl.ds(p * 32, 16)]
                sb = s_ref[pl.ds(p * 32 + 16, 16)]
                db = d_ref[pl.ds(p * 32 + 16, 16)]
                _, la = plsc.scan_count(sa)
                _, lb = plsc.scan_count(sb)
                rmw(mas, sa, da)
                rmw(mbs, sb, db)
                anydup = jnp.any(jnp.logical_not(la & lb))

                @pl.when(anydup)
                def _():
                    fixup(mas, sa, da)
                    fixup(mbs, sb, db)

                return carry

            lax.fori_loop(0, G // 2, pair_body, 0)

        def start_chunk(c, s_buf, d_buf, sem):
            pltpu.make_async_copy(s_hbm.at[pl.ds(c * CH, CH)], s_buf,
                                  sem).start()
            pltpu.make_async_copy(d_hbm.at[pl.ds(c * CH, CH)], d_buf,
                                  sem).start()

        def wait_chunk(c, s_buf, d_buf, sem):
            pltpu.make_async_copy(s_hbm.at[pl.ds(c * CH, CH)], s_buf,
                                  sem).wait()
            pltpu.make_async_copy(d_hbm.at[pl.ds(c * CH, CH)], d_buf,
                                  sem).wait()

        start_chunk(0, s0, d0, sem0)

        def outer(i, carry):
            c0 = 2 * i
            start_chunk(c0 + 1, s1, d1, sem1)
            wait_chunk(c0, s0, d0, sem0)
            process(s0, d0)

            @pl.when(c0 + 2 < NCH)
            def _():
                start_chunk(c0 + 2, s0, d0, sem0)

            wait_chunk(c0 + 1, s1, d1, sem1)
            process(s1, d1)
            return carry

        lax.fori_loop(0, NCH // 2, outer, 0)

        def merge_body(i, c):
            for f in range(FPW):
                mas[f][pl.ds(i * 16, 16)] = jnp.maximum(
                    mas[f][pl.ds(i * 16, 16)], mbs[f][pl.ds(i * 16, 16)])
            return c

        lax.fori_loop(0, N // 16, merge_body, 0)
        for f in range(FPW):
            pltpu.sync_copy(mas[f], out_hbm.at[pl.ds(base + f * N, N)])

    return k(bT_flat, src, dst)


def kernel(child_feats, child_exists, edge_indices,
           W_m1a, b_m1a, W_m1b, b_m1b, W_skip10, b_skip10, W_m2, b_m2,
           W_child, b_child, W_ne0, b_ne0, W_ne1, b_ne1,
           W_skipobj, b_skipobj, W_second, b_second):
    N = child_feats.shape[1]
    H = 128
    Np = ((N + _BN - 1) // _BN) * _BN

    xT = child_feats[0].T
    pad = ((0, 0), (0, Np - N))
    boxT = jnp.pad(xT[:10], pad)
    semT = jnp.pad(xT[10:], pad)
    ceT = jnp.pad(child_exists[0].T, pad)
    src = edge_indices[0, :, 0]
    dst = edge_indices[0, :, 1]

    colv = lambda b: b.reshape(-1, 1)
    ws = [W_m1a.T, colv(b_m1a), W_m1b.T, colv(b_m1b),
          W_skip10.T, colv(b_skip10), W_m2.T, colv(b_m2),
          W_child.T, colv(b_child), W_skipobj.T, colv(b_skipobj),
          W_ne0[:H].T, W_ne0[H:].T]

    A1T, B1T, p1, skmax = _run_k1(boxT, semT, ceT, ws, Np)

    M1T = _segmax_sc(B1T.reshape(-1), src, dst, Np).reshape(H, Np)
    A2T, B2T, p2 = _run_k2(A1T, M1T, colv(b_ne0),
                           W_ne1[:H].T, W_ne1[H:].T, Np)
    M2T = _segmax_sc(B2T.reshape(-1), src, dst, Np).reshape(H, Np)
    outT = _run_k3(A2T, M2T, colv(b_ne1), p1, p2, skmax,
                   W_second.T, colv(b_second), Np)
    return outT.reshape(1, H)


# speedup vs baseline: 3.7348x; 1.2281x over previous
import functools

import jax
import jax.numpy as jnp
from jax import lax
from jax.experimental import pallas as pl
from jax.experimental.pallas import tpu as pltpu
from jax.experimental.pallas import tpu_sc as plsc

_NEG = -3.0e38
_BN = 2048


def _lr(x):
    return jnp.where(x > 0, x, 0.1 * x)


def _k1_body(boxT, semT, ceT, Wm1aT, bm1a, Wm1bT, bm1b, Ws10T, bs10,
             Wm2T, bm2, WchT, bch, WsoT, bso, WsrcT, WdstT,
             A1T_o, B1T_o, p1_o, sk_o):
    j = pl.program_id(0)
    box = boxT[...]
    h = _lr(Wm1aT[...] @ box + bm1a[...])
    h = _lr(Wm1bT[...] @ h + bm1b[...])
    enc = _lr(Ws10T[...] @ box + bs10[...] + Wm2T[...] @ h + bm2[...])
    cfeat = jnp.concatenate([enc, semT[...]], axis=0)
    ce = ceT[...]
    skip = jnp.where(ce > 0, (WsoT[...] @ cfeat + bso[...]) * ce, _NEG)
    cf1 = jnp.maximum(WchT[...] @ cfeat + bch[...], 0.0) * ce
    A1T_o[...] = WsrcT[...] @ cf1
    B1T_o[...] = WdstT[...] @ cf1
    pb = jnp.max(cf1, axis=1, keepdims=True)
    sb = jnp.max(skip, axis=1, keepdims=True)

    @pl.when(j == 0)
    def _():
        p1_o[...] = pb
        sk_o[...] = sb

    @pl.when(j > 0)
    def _():
        p1_o[...] = jnp.maximum(p1_o[...], pb)
        sk_o[...] = jnp.maximum(sk_o[...], sb)


def _run_k1(boxT, semT, ceT, ws, N):
    ng = N // _BN
    col = lambda r: pl.BlockSpec((r, _BN), lambda j: (0, j))
    full = lambda a: pl.BlockSpec(a.shape, lambda j: (0, 0))
    out_col = pl.BlockSpec((128, _BN), lambda j: (0, j))
    acc = pl.BlockSpec((128, 1), lambda j: (0, 0))
    return pl.pallas_call(
        _k1_body,
        grid=(ng,),
        in_specs=[col(10), col(24), col(1)] + [full(w) for w in ws],
        out_specs=[out_col, out_col, acc, acc],
        out_shape=[
            jax.ShapeDtypeStruct((128, N), jnp.float32),
            jax.ShapeDtypeStruct((128, N), jnp.float32),
            jax.ShapeDtypeStruct((128, 1), jnp.float32),
            jax.ShapeDtypeStruct((128, 1), jnp.float32),
        ],
    )(boxT, semT, ceT, *ws)


def _k2_body(AT, MT, bne, WsrcT, WdstT, A2T_o, B2T_o, p_o):
    j = pl.program_id(0)
    cf = jnp.maximum(AT[...] + bne[...] + MT[...], 0.0)
    A2T_o[...] = WsrcT[...] @ cf
    B2T_o[...] = WdstT[...] @ cf
    pb = jnp.max(cf, axis=1, keepdims=True)

    @pl.when(j == 0)
    def _():
        p_o[...] = pb

    @pl.when(j > 0)
    def _():
        p_o[...] = jnp.maximum(p_o[...], pb)


def _run_k2(AT, MT, bne, WsrcT, WdstT, N):
    ng = N // _BN
    col = pl.BlockSpec((128, _BN), lambda j: (0, j))
    full = lambda a: pl.BlockSpec(a.shape, lambda j: (0, 0))
    acc = pl.BlockSpec((128, 1), lambda j: (0, 0))
    return pl.pallas_call(
        _k2_body,
        grid=(ng,),
        in_specs=[col, col, full(bne), full(WsrcT), full(WdstT)],
        out_specs=[col, col, acc],
        out_shape=[
            jax.ShapeDtypeStruct((128, N), jnp.float32),
            jax.ShapeDtypeStruct((128, N), jnp.float32),
            jax.ShapeDtypeStruct((128, 1), jnp.float32),
        ],
    )(AT, MT, bne, WsrcT, WdstT)


def _k3_body(AT, MT, bne, p1, p2, skmax, WsecT, bsec, out_o, p3s):
    j = pl.program_id(0)
    ng = pl.num_programs(0)
    cf = jnp.maximum(AT[...] + bne[...] + MT[...], 0.0)
    pb = jnp.max(cf, axis=1, keepdims=True)

    @pl.when(j == 0)
    def _():
        p3s[...] = pb

    @pl.when(j > 0)
    def _():
        p3s[...] = jnp.maximum(p3s[...], pb)

    @pl.when(j == ng - 1)
    def _():
        parent = jnp.concatenate([p1[...], p2[...], p3s[...]], axis=0)
        out_o[...] = _lr(_lr(skmax[...]) + WsecT[...] @ parent + bsec[...])


def _run_k3(AT, MT, bne, p1, p2, skmax, WsecT, bsec, N):
    ng = N // _BN
    col = pl.BlockSpec((128, _BN), lambda j: (0, j))
    full = lambda a: pl.BlockSpec(a.shape, lambda j: (0, 0))
    acc = pl.BlockSpec((128, 1), lambda j: (0, 0))
    return pl.pallas_call(
        _k3_body,
        grid=(ng,),
        in_specs=[col, col, full(bne), full(p1), full(p2), full(skmax),
                  full(WsecT), full(bsec)],
        out_specs=acc,
        out_shape=jax.ShapeDtypeStruct((128, 1), jnp.float32),
        scratch_shapes=[pltpu.VMEM((128, 1), jnp.float32)],
    )(AT, MT, bne, p1, p2, skmax, WsecT, bsec)


def _segmax_sc(bT_flat, src, dst, N):
    E = src.shape[0]
    NC, NS = 2, 16
    NW = NC * NS
    FPW = 128 // NW
    CH = 1600
    G = CH // 16
    NCH = E // CH
    assert CH * NCH == E and FPW * NW == 128
    assert G % 2 == 0 and NCH % 2 == 0

    mesh = plsc.VectorSubcoreMesh(core_axis_name="c", subcore_axis_name="s",
                                  num_cores=NC, num_subcores=NS)

    @functools.partial(
        pl.kernel,
        out_type=jax.ShapeDtypeStruct((128 * N,), jnp.float32),
        mesh=mesh,
        scratch_types=(
            [pltpu.VMEM((N,), jnp.float32)] * FPW +
            [pltpu.VMEM((N,), jnp.float32)] * FPW +
            [pltpu.VMEM((N,), jnp.float32)] * FPW +
            [
                pltpu.VMEM((CH,), jnp.int32),
                pltpu.VMEM((CH,), jnp.int32),
                pltpu.VMEM((CH,), jnp.int32),
                pltpu.VMEM((CH,), jnp.int32),
                pltpu.SemaphoreType.DMA,
                pltpu.SemaphoreType.DMA,
            ]
        ),
        compiler_params=pltpu.CompilerParams(needs_layout_passes=False),
    )
    def k(b_hbm, s_hbm, d_hbm, out_hbm,
          b0, b1, b2, b3, ma0, ma1, ma2, ma3, mb0, mb1, mb2, mb3,
          s0, d0, s1, d1, sem0, sem1):
        bs = [b0, b1, b2, b3]
        mas = [ma0, ma1, ma2, ma3]
        mbs = [mb0, mb1, mb2, mb3]
        wid = lax.axis_index("s") * NC + lax.axis_index("c")
        base = wid * (FPW * N)
        for f in range(FPW):
            pltpu.sync_copy(b_hbm.at[pl.ds(base + f * N, N)], bs[f])

        neg = jnp.full((16,), _NEG, jnp.float32)

        def init_body(i, c):
            for f in range(FPW):
                mas[f][pl.ds(i * 16, 16)] = neg
                mbs[f][pl.ds(i * 16, 16)] = neg
            return c

        lax.fori_loop(0, N // 16, init_body, 0)

        lanes = lax.broadcasted_iota(jnp.int32, (16,), 0)

        def rmw_pair(sa, da, sb, db):
            va = [plsc.load_gather(bs[f], [da]) for f in range(FPW)]
            vb = [plsc.load_gather(bs[f], [db]) for f in range(FPW)]
            oa = [plsc.load_gather(mas[f], [sa]) for f in range(FPW)]
            ob = [plsc.load_gather(mbs[f], [sb]) for f in range(FPW)]
            for f in range(FPW):
                plsc.store_scatter(mas[f], [sa], jnp.maximum(oa[f], va[f]))
            for f in range(FPW):
                plsc.store_scatter(mbs[f], [sb], jnp.maximum(ob[f], vb[f]))

        def fixup(ms, s16, d16):
            vs = [plsc.load_gather(bs[f], [d16]) for f in range(FPW)]

            def lane_body(l, c3):
                m = lanes == l
                for f in range(FPW):
                    old = plsc.load_gather(ms[f], [s16], mask=m)
                    plsc.store_scatter(ms[f], [s16],
                                       jnp.maximum(old, vs[f]), mask=m)
                return c3

            lax.fori_loop(0, 16, lane_body, 0)

        def process(s_ref, d_ref):
            def pair_body(p, carry):
                sa = s_ref[pl.ds(p * 32, 16)]
                da = d_ref[pl.ds(p * 32, 16)]
                sb = s_ref[pl.ds(p * 32 + 16, 16)]
                db = d_ref[pl.ds(p * 32 + 16, 16)]
                _, la = plsc.scan_count(sa)
                _, lb = plsc.scan_count(sb)
                rmw_pair(sa, da, sb, db)
                anydup = jnp.any(jnp.logical_not(la & lb))

                @pl.when(anydup)
                def _():
                    fixup(mas, sa, da)
                    fixup(mbs, sb, db)

                return carry

            lax.fori_loop(0, G // 2, pair_body, 0)

        def start_chunk(c, s_buf, d_buf, sem):
            pltpu.make_async_copy(s_hbm.at[pl.ds(c * CH, CH)], s_buf,
                                  sem).start()
            pltpu.make_async_copy(d_hbm.at[pl.ds(c * CH, CH)], d_buf,
                                  sem).start()

        def wait_chunk(c, s_buf, d_buf, sem):
            pltpu.make_async_copy(s_hbm.at[pl.ds(c * CH, CH)], s_buf,
                                  sem).wait()
            pltpu.make_async_copy(d_hbm.at[pl.ds(c * CH, CH)], d_buf,
                                  sem).wait()

        start_chunk(0, s0, d0, sem0)

        def outer(i, carry):
            c0 = 2 * i
            start_chunk(c0 + 1, s1, d1, sem1)
            wait_chunk(c0, s0, d0, sem0)
            process(s0, d0)

            @pl.when(c0 + 2 < NCH)
            def _():
                start_chunk(c0 + 2, s0, d0, sem0)

            wait_chunk(c0 + 1, s1, d1, sem1)
            process(s1, d1)
            return carry

        lax.fori_loop(0, NCH // 2, outer, 0)

        def merge_body(i, c):
            for f in range(FPW):
                mas[f][pl.ds(i * 16, 16)] = jnp.maximum(
                    mas[f][pl.ds(i * 16, 16)], mbs[f][pl.ds(i * 16, 16)])
            return c

        lax.fori_loop(0, N // 16, merge_body, 0)
        for f in range(FPW):
            pltpu.sync_copy(mas[f], out_hbm.at[pl.ds(base + f * N, N)])

    return k(bT_flat, src, dst)


def kernel(child_feats, child_exists, edge_indices,
           W_m1a, b_m1a, W_m1b, b_m1b, W_skip10, b_skip10, W_m2, b_m2,
           W_child, b_child, W_ne0, b_ne0, W_ne1, b_ne1,
           W_skipobj, b_skipobj, W_second, b_second):
    N = child_feats.shape[1]
    H = 128
    Np = ((N + _BN - 1) // _BN) * _BN

    xT = child_feats[0].T
    pad = ((0, 0), (0, Np - N))
    boxT = jnp.pad(xT[:10], pad)
    semT = jnp.pad(xT[10:], pad)
    ceT = jnp.pad(child_exists[0].T, pad)
    src = edge_indices[0, :, 0]
    dst = edge_indices[0, :, 1]

    colv = lambda b: b.reshape(-1, 1)
    ws = [W_m1a.T, colv(b_m1a), W_m1b.T, colv(b_m1b),
          W_skip10.T, colv(b_skip10), W_m2.T, colv(b_m2),
          W_child.T, colv(b_child), W_skipobj.T, colv(b_skipobj),
          W_ne0[:H].T, W_ne0[H:].T]

    A1T, B1T, p1, skmax = _run_k1(boxT, semT, ceT, ws, Np)

    M1T = _segmax_sc(B1T.reshape(-1), src, dst, Np).reshape(H, Np)
    A2T, B2T, p2 = _run_k2(A1T, M1T, colv(b_ne0),
                           W_ne1[:H].T, W_ne1[H:].T, Np)
    M2T = _segmax_sc(B2T.reshape(-1), src, dst, Np).reshape(H, Np)
    outT = _run_k3(A2T, M2T, colv(b_ne1), p1, p2, skmax,
                   W_second.T, colv(b_second), Np)
    return outT.reshape(1, H)
